# Initial kernel scaffold; baseline (speedup 1.0000x reference)
import jax
import jax.numpy as jnp
from jax.experimental import pallas as pl


def kernel(concept_ids, emb_table, vertex_emb, enc_w, enc_b, enc_ln_g, enc_ln_b, dec_w, dec_b, dec_ln_g, dec_ln_b, ar_w1, ar_b1, ar_ln1_g, ar_ln1_b, ar_w2, ar_b2, ar_ln2_g, ar_ln2_b):
    raise NotImplementedError("write your pallas kernel here")


# speedup vs baseline: 1.7065x; 1.7065x over previous
import functools

import jax
import jax.numpy as jnp
from jax import lax
from jax.experimental import pallas as pl
from jax.experimental.pallas import tpu as pltpu
from jax.experimental.pallas import tpu_sc as plsc

D = 256
K = 4096
N_TOK = 8 * 576
BLK = 512
NW = 32
PER_W = N_TOK // NW
CHUNK = 72
NCHUNK = PER_W // CHUNK


def _sc_gather(table, idx2d):
    mesh = plsc.VectorSubcoreMesh(core_axis_name="c", subcore_axis_name="s")

    @functools.partial(
        pl.kernel,
        mesh=mesh,
        out_type=jax.ShapeDtypeStruct((N_TOK, D), jnp.float32),
        scratch_types=[
            pltpu.VMEM((NCHUNK, CHUNK), jnp.int32),
            pltpu.VMEM((NCHUNK, CHUNK, D), jnp.float32),
            pltpu.SemaphoreType.DMA,
        ],
    )
    def k(table_hbm, idx_hbm, out_hbm, idx_v, rows_v, sem):
        wid = lax.axis_index("s") * 2 + lax.axis_index("c")
        base = wid * PER_W
        pltpu.sync_copy(idx_hbm.at[wid], idx_v)
        copies = []
        for j in range(NCHUNK):
            copies.append(
                pltpu.async_copy(table_hbm.at[idx_v.at[j]], rows_v.at[j], sem))
        for c in copies:
            c.wait()
        for j in range(NCHUNK):
            pltpu.sync_copy(rows_v.at[j],
                            out_hbm.at[pl.ds(base + j * CHUNK, CHUNK)])

    return k(table, idx2d)


def _gelu(x):
    return x * 0.5 * (1.0 + lax.erf(x * 0.7071067811865476))


def _ln(x, g, b):
    m = jnp.mean(x, axis=-1, keepdims=True)
    v = jnp.mean((x - m) ** 2, axis=-1, keepdims=True)
    return (x - m) * lax.rsqrt(v + 1e-5) * g + b


def _vprep_body(v_ref, vhi_ref, vlo_ref, v2_ref):
    v = v_ref[...]
    vh = v.astype(jnp.bfloat16)
    vhi_ref[...] = vh
    vlo_ref[...] = (v - vh.astype(jnp.float32)).astype(jnp.bfloat16)
    v2_ref[...] = jnp.sum(v * v, axis=1)[None, :]


def _vprep(vertex_emb, interpret=False):
    return pl.pallas_call(
        _vprep_body,
        out_shape=(
            jax.ShapeDtypeStruct((K, D), jnp.bfloat16),
            jax.ShapeDtypeStruct((K, D), jnp.bfloat16),
            jax.ShapeDtypeStruct((1, K), jnp.float32),
        ),
        interpret=interpret,
    )(vertex_emb)


def _tc_body(raw_ref, vhi_ref, vlo_ref, v2_ref, ew_ref, eb_ref, eg_ref,
             ebb_ref, dw_ref, db_ref, dg_ref, dbb_ref,
             w1_ref, b1_ref, g1_ref, bb1_ref,
             w2_ref, b2_ref, g2_ref, bb2_ref,
             disc_ref, cont_ref, pred_ref, racc_ref):
    i = pl.program_id(0)
    x = raw_ref[...]

    xm2 = (x * -2.0).astype(jnp.bfloat16)
    s = lax.dot_general(xm2, vhi_ref[...], (((1,), (1,)), ((), ())),
                        preferred_element_type=jnp.float32)
    s = s + v2_ref[...]
    nearest = jnp.argmin(s, axis=1)

    onehot = jnp.where(
        lax.broadcasted_iota(jnp.int16, (BLK, K), 1)
        == nearest.astype(jnp.int16)[:, None],
        jnp.bfloat16(1.0), jnp.bfloat16(0.0))
    disc = (lax.dot_general(onehot, vhi_ref[...], (((1,), (0,)), ((), ())),
                            preferred_element_type=jnp.float32)
            + lax.dot_general(onehot, vlo_ref[...], (((1,), (0,)), ((), ())),
                              preferred_element_type=jnp.float32))
    disc_ref[...] = disc

    def mm_t(a, w):
        return lax.dot_general(a.astype(jnp.bfloat16), w.astype(jnp.bfloat16),
                               (((1,), (1,)), ((), ())),
                               preferred_element_type=jnp.float32)

    cont = _gelu(_ln(mm_t(disc, ew_ref[...]) + eb_ref[...],
                     eg_ref[...], ebb_ref[...]))
    cont_ref[...] = cont

    rec = _gelu(_ln(mm_t(cont, dw_ref[...]) + db_ref[...],
                    dg_ref[...], dbb_ref[...]))
    part = jnp.sum((rec - disc) ** 2)

    @pl.when(i == 0)
    def _():
        racc_ref[...] = jnp.zeros_like(racc_ref)

    racc_ref[...] += part[None, None]

    w1s = w1_ref[:, :D] + w1_ref[:, D:]
    h = _gelu(mm_t(cont, w1s) + b1_ref[...])
    h = _ln(h, g1_ref[...], bb1_ref[...])
    h = _gelu(mm_t(h, w2_ref[...]) + b2_ref[...])
    pred_ref[...] = _ln(h, g2_ref[...], bb2_ref[...])


def _tc_forward(raw, vhi, vlo, v2, enc_w, enc_b, enc_ln_g, enc_ln_b,
                dec_w, dec_b, dec_ln_g, dec_ln_b,
                ar_w1, ar_b1, ar_ln1_g, ar_ln1_b,
                ar_w2, ar_b2, ar_ln2_g, ar_ln2_b, interpret=False):
    H = 2 * D
    row = lambda n: pl.BlockSpec((1, n), lambda i: (0, 0))
    full = lambda m, n: pl.BlockSpec((m, n), lambda i: (0, 0))
    blk = pl.BlockSpec((BLK, D), lambda i: (i, 0))
    grid = (N_TOK // BLK,)
    out_shapes = (
        jax.ShapeDtypeStruct((N_TOK, D), jnp.float32),
        jax.ShapeDtypeStruct((N_TOK, D), jnp.float32),
        jax.ShapeDtypeStruct((N_TOK, D), jnp.float32),
        jax.ShapeDtypeStruct((1, 1), jnp.float32),
    )
    return pl.pallas_call(
        _tc_body,
        grid=grid,
        in_specs=[
            blk,
            full(K, D),
            full(K, D),
            full(1, K),
            full(D, D), row(D), row(D), row(D),
            full(D, D), row(D), row(D), row(D),
            full(H, H), row(H), row(H), row(H),
            full(D, H), row(D), row(D), row(D),
        ],
        out_specs=(blk, blk, blk, pl.BlockSpec((1, 1), lambda i: (0, 0))),
        out_shape=out_shapes,
        interpret=interpret,
    )(raw, vhi, vlo, v2,
      enc_w, enc_b.reshape(1, D), enc_ln_g.reshape(1, D), enc_ln_b.reshape(1, D),
      dec_w, dec_b.reshape(1, D), dec_ln_g.reshape(1, D), dec_ln_b.reshape(1, D),
      ar_w1, ar_b1.reshape(1, H), ar_ln1_g.reshape(1, H), ar_ln1_b.reshape(1, H),
      ar_w2, ar_b2.reshape(1, D), ar_ln2_g.reshape(1, D), ar_ln2_b.reshape(1, D))


def kernel(concept_ids, emb_table, vertex_emb, enc_w, enc_b, enc_ln_g, enc_ln_b,
           dec_w, dec_b, dec_ln_g, dec_ln_b,
           ar_w1, ar_b1, ar_ln1_g, ar_ln1_b, ar_w2, ar_b2, ar_ln2_g, ar_ln2_b):
    B, T = concept_ids.shape
    idx2d = concept_ids.reshape(NW, NCHUNK, CHUNK).astype(jnp.int32)
    raw = _sc_gather(emb_table, idx2d)
    vhi, vlo, v2 = _vprep(vertex_emb)
    disc, cont, pred, rsum = _tc_forward(
        raw, vhi, vlo, v2, enc_w, enc_b, enc_ln_g, enc_ln_b,
        dec_w, dec_b, dec_ln_g, dec_ln_b,
        ar_w1, ar_b1, ar_ln1_g, ar_ln1_b, ar_w2, ar_b2, ar_ln2_g, ar_ln2_b)
    disc = disc.reshape(B, T, D)
    cont = cont.reshape(B, T, D)
    pred = pred.reshape(B, T, D)
    recon_loss = (rsum[0, 0] / (N_TOK * D)).astype(jnp.float32)
    energy_loss = jnp.zeros((), jnp.float32)
    return (disc, cont, pred, recon_loss, energy_loss)


# speedup vs baseline: 1.8402x; 1.0783x over previous
import functools

import jax
import jax.numpy as jnp
from jax import lax
from jax.experimental import pallas as pl
from jax.experimental.pallas import tpu as pltpu
from jax.experimental.pallas import tpu_sc as plsc

D = 256
K = 4096
N_TOK = 8 * 576
BLK = 512
NW = 32
PER_W = N_TOK // NW
CHUNK = 72
NCHUNK = PER_W // CHUNK


def _sc_gather(table, idx2d):
    mesh = plsc.VectorSubcoreMesh(core_axis_name="c", subcore_axis_name="s")

    @functools.partial(
        pl.kernel,
        mesh=mesh,
        out_type=jax.ShapeDtypeStruct((N_TOK, D), jnp.float32),
        scratch_types=[
            pltpu.VMEM((NCHUNK, CHUNK), jnp.int32),
            pltpu.VMEM((NCHUNK, CHUNK, D), jnp.float32),
            pltpu.SemaphoreType.DMA,
        ],
    )
    def k(table_hbm, idx_hbm, out_hbm, idx_v, rows_v, sem):
        wid = lax.axis_index("s") * 2 + lax.axis_index("c")
        base = wid * PER_W
        pltpu.sync_copy(idx_hbm.at[wid], idx_v)
        copies = []
        for j in range(NCHUNK):
            copies.append(
                pltpu.async_copy(table_hbm.at[idx_v.at[j]], rows_v.at[j], sem))
        for c in copies:
            c.wait()
        for j in range(NCHUNK):
            pltpu.sync_copy(rows_v.at[j],
                            out_hbm.at[pl.ds(base + j * CHUNK, CHUNK)])

    return k(table, idx2d)


def _gelu(x):
    return x * 0.5 * (1.0 + lax.erf(x * 0.7071067811865476))


def _ln(x, g, b):
    m = jnp.mean(x, axis=-1, keepdims=True)
    v = jnp.mean((x - m) ** 2, axis=-1, keepdims=True)
    return (x - m) * lax.rsqrt(v + 1e-5) * g + b


def _vprep_body(v_ref, vhi_ref, vlo_ref, v2_ref):
    v = v_ref[...]
    vh = v.astype(jnp.bfloat16)
    vhi_ref[...] = vh
    vlo_ref[...] = (v - vh.astype(jnp.float32)).astype(jnp.bfloat16)
    v2_ref[...] = jnp.sum(v * v, axis=1)[None, :]


def _vprep(vertex_emb, interpret=False):
    return pl.pallas_call(
        _vprep_body,
        out_shape=(
            jax.ShapeDtypeStruct((K, D), jnp.bfloat16),
            jax.ShapeDtypeStruct((K, D), jnp.bfloat16),
            jax.ShapeDtypeStruct((1, K), jnp.float32),
        ),
        interpret=interpret,
    )(vertex_emb)


def _quant_body(raw_ref, vhi_ref, vlo_ref, v2_ref, disc_ref):
    x = raw_ref[...]
    xm2 = (x * -2.0).astype(jnp.bfloat16)
    s = lax.dot_general(xm2, vhi_ref[...], (((1,), (1,)), ((), ())),
                        preferred_element_type=jnp.float32)
    s = s + v2_ref[...]
    nearest = jnp.argmin(s, axis=1)
    onehot = jnp.where(
        lax.broadcasted_iota(jnp.int16, (BLK, K), 1)
        == nearest.astype(jnp.int16)[:, None],
        jnp.bfloat16(1.0), jnp.bfloat16(0.0))
    disc_ref[...] = (
        lax.dot_general(onehot, vhi_ref[...], (((1,), (0,)), ((), ())),
                        preferred_element_type=jnp.float32)
        + lax.dot_general(onehot, vlo_ref[...], (((1,), (0,)), ((), ())),
                          preferred_element_type=jnp.float32))


def _quantize(raw, vhi, vlo, v2, interpret=False):
    return pl.pallas_call(
        _quant_body,
        grid=(N_TOK // BLK,),
        in_specs=[
            pl.BlockSpec((BLK, D), lambda i: (i, 0)),
            pl.BlockSpec((K, D), lambda i: (0, 0)),
            pl.BlockSpec((K, D), lambda i: (0, 0)),
            pl.BlockSpec((1, K), lambda i: (0, 0)),
        ],
        out_specs=pl.BlockSpec((BLK, D), lambda i: (i, 0)),
        out_shape=jax.ShapeDtypeStruct((N_TOK, D), jnp.float32),
        interpret=interpret,
    )(raw, vhi, vlo, v2)


def _mlp_body(disc_ref, ew_ref, eb_ref, eg_ref, ebb_ref,
              dw_ref, db_ref, dg_ref, dbb_ref,
              w1_ref, b1_ref, g1_ref, bb1_ref,
              w2_ref, b2_ref, g2_ref, bb2_ref,
              cont_ref, pred_ref, racc_ref):
    disc = disc_ref[...]

    def mm_t(a, w):
        return lax.dot_general(a.astype(jnp.bfloat16), w.astype(jnp.bfloat16),
                               (((1,), (1,)), ((), ())),
                               preferred_element_type=jnp.float32)

    cont = _gelu(_ln(mm_t(disc, ew_ref[...]) + eb_ref[...],
                     eg_ref[...], ebb_ref[...]))
    cont_ref[...] = cont

    rec = _gelu(_ln(mm_t(cont, dw_ref[...]) + db_ref[...],
                    dg_ref[...], dbb_ref[...]))
    racc_ref[...] = jnp.sum((rec - disc) ** 2)[None, None]

    w1s = w1_ref[:, :D] + w1_ref[:, D:]
    h = _gelu(mm_t(cont, w1s) + b1_ref[...])
    h = _ln(h, g1_ref[...], bb1_ref[...])
    h = _gelu(mm_t(h, w2_ref[...]) + b2_ref[...])
    pred_ref[...] = _ln(h, g2_ref[...], bb2_ref[...])


def _mlp(disc, enc_w, enc_b, enc_ln_g, enc_ln_b,
         dec_w, dec_b, dec_ln_g, dec_ln_b,
         ar_w1, ar_b1, ar_ln1_g, ar_ln1_b,
         ar_w2, ar_b2, ar_ln2_g, ar_ln2_b, interpret=False):
    H = 2 * D
    return pl.pallas_call(
        _mlp_body,
        out_shape=(
            jax.ShapeDtypeStruct((N_TOK, D), jnp.float32),
            jax.ShapeDtypeStruct((N_TOK, D), jnp.float32),
            jax.ShapeDtypeStruct((1, 1), jnp.float32),
        ),
        interpret=interpret,
    )(disc,
      enc_w, enc_b.reshape(1, D), enc_ln_g.reshape(1, D), enc_ln_b.reshape(1, D),
      dec_w, dec_b.reshape(1, D), dec_ln_g.reshape(1, D), dec_ln_b.reshape(1, D),
      ar_w1, ar_b1.reshape(1, H), ar_ln1_g.reshape(1, H), ar_ln1_b.reshape(1, H),
      ar_w2, ar_b2.reshape(1, D), ar_ln2_g.reshape(1, D), ar_ln2_b.reshape(1, D))


def kernel(concept_ids, emb_table, vertex_emb, enc_w, enc_b, enc_ln_g, enc_ln_b,
           dec_w, dec_b, dec_ln_g, dec_ln_b,
           ar_w1, ar_b1, ar_ln1_g, ar_ln1_b, ar_w2, ar_b2, ar_ln2_g, ar_ln2_b):
    B, T = concept_ids.shape
    idx2d = concept_ids.reshape(NW, NCHUNK, CHUNK).astype(jnp.int32)
    raw = _sc_gather(emb_table, idx2d)
    vhi, vlo, v2 = _vprep(vertex_emb)
    disc = _quantize(raw, vhi, vlo, v2)
    cont, pred, rsum = _mlp(
        disc, enc_w, enc_b, enc_ln_g, enc_ln_b,
        dec_w, dec_b, dec_ln_g, dec_ln_b,
        ar_w1, ar_b1, ar_ln1_g, ar_ln1_b, ar_w2, ar_b2, ar_ln2_g, ar_ln2_b)
    recon_loss = (rsum[0, 0] / (N_TOK * D)).astype(jnp.float32)
    energy_loss = jnp.zeros((), jnp.float32)
    return (disc.reshape(B, T, D), cont.reshape(B, T, D),
            pred.reshape(B, T, D), recon_loss, energy_loss)
